# baseline (speedup 1.0000x reference)
import functools

import jax
import jax.numpy as jnp
from jax import lax
from jax.experimental import pallas as pl
from jax.experimental.pallas import tpu as pltpu
from jax.experimental.pallas import tpu_sc as plsc

_NC = 2
_NS = 16
_NW = _NC * _NS
_L = 16


def _silu(x):
    return x * lax.logistic(x)


def _proj_body(hq_ref, hkv_ref, wq_ref, wk_ref, be1_ref, aq_ref, ak_ref):
    aq_ref[...] = jnp.dot(hq_ref[...], wq_ref[...],
                          preferred_element_type=jnp.float32)
    ak_ref[...] = jnp.dot(hkv_ref[...], wk_ref[...],
                          preferred_element_type=jnp.float32) + be1_ref[...]


_NBUF = 4
_GBUF = 5
_GLEAD = 3


@functools.lru_cache(maxsize=None)
def _make_gather_add(N, H, E):
    epw = E // _NW
    C = 80
    nch = epw // C
    ngrp = nch // _GBUF
    assert nch == ngrp * _GBUF
    mesh = plsc.VectorSubcoreMesh(core_axis_name="c", subcore_axis_name="s")

    @functools.partial(
        pl.kernel,
        out_type=jax.ShapeDtypeStruct((E, H), jnp.float32),
        mesh=mesh,
        scratch_types=[
            pltpu.VMEM((epw,), jnp.int32),
            pltpu.VMEM((epw,), jnp.int32),
            [pltpu.VMEM((C, H), jnp.float32)] * _GBUF,
            [pltpu.VMEM((C, H), jnp.float32)] * _GBUF,
            [pltpu.SemaphoreType.DMA] * _GBUF,
            [pltpu.SemaphoreType.DMA] * _GBUF,
        ],
    )
    def gather_add(aq_hbm, ak_hbm, row_hbm, col_hbm, out_hbm,
                   ridx, cidx, bqs, bks, gsems, osems):
        wid = lax.axis_index("s") * _NC + lax.axis_index("c")
        base = wid * epw

        pltpu.sync_copy(row_hbm.at[pl.ds(base, epw)], ridx)
        pltpu.sync_copy(col_hbm.at[pl.ds(base, epw)], cidx)

        def issue_gather(k, b):
            pltpu.async_copy(aq_hbm.at[ridx.at[pl.ds(k * C, C)]], bqs[b],
                             gsems[b])
            pltpu.async_copy(ak_hbm.at[cidx.at[pl.ds(k * C, C)]], bks[b],
                             gsems[b])

        def wait_gather(k, b):
            pltpu.make_async_copy(aq_hbm.at[ridx.at[pl.ds(k * C, C)]],
                                  bqs[b], gsems[b]).wait()
            pltpu.make_async_copy(ak_hbm.at[cidx.at[pl.ds(k * C, C)]],
                                  bks[b], gsems[b]).wait()

        def add_and_out(k, b):
            bq, bk = bqs[b], bks[b]

            def add_row(e, c2):
                for j in range(H // _L):
                    sl = pl.ds(j * _L, _L)
                    bq[e, sl] = bq[e, sl] + bk[e, sl]
                return c2

            lax.fori_loop(0, C, add_row, 0, unroll=2)
            pltpu.async_copy(bq, out_hbm.at[pl.ds(base + k * C, C)],
                             osems[b])

        def wait_out(b):
            pltpu.make_async_copy(bqs[b], out_hbm.at[pl.ds(base, C)],
                                  osems[b]).wait()

        for k0 in range(_GLEAD):
            issue_gather(k0, k0)

        def group(i, c):
            for p in range(_GBUF):
                k = _GBUF * i + p
                nk = k + _GLEAD
                nb = (p + _GLEAD) % _GBUF
                if p < _GLEAD - 1:
                    @pl.when(i > 0)
                    def _():
                        wait_out(nb)
                    issue_gather(nk, nb)
                else:
                    @pl.when(i < ngrp - 1)
                    def _():
                        wait_out(nb)
                        issue_gather(nk, nb)
                wait_gather(k, p)
                add_and_out(k, p)
            return c

        lax.fori_loop(0, ngrp, group, 0)
        for b in range(_GBUF):
            wait_out(b)

    return gather_add


def _edge_body(g_ref, w2_ref, b2_ref, out_ref):
    h1 = _silu(g_ref[...])
    out_ref[...] = _silu(jnp.dot(h1, w2_ref[...],
                                 preferred_element_type=jnp.float32)
                         + b2_ref[...])


@functools.lru_cache(maxsize=None)
def _make_scatter_add(N, H, E):
    epw = E // _NW
    C = 80
    nch = epw // C
    n_pad = ((N + _NS * 8 - 1) // (_NS * 8)) * (_NS * 8)
    rpt = n_pad // _NS
    mesh = plsc.VectorSubcoreMesh(core_axis_name="c", subcore_axis_name="s")

    ngrp = (nch - 1) // _NBUF
    assert nch == 1 + ngrp * _NBUF

    @functools.partial(
        pl.kernel,
        out_type=jax.ShapeDtypeStruct((_NC, n_pad, H), jnp.float32),
        mesh=mesh,
        scratch_types=[
            [pltpu.VMEM((C,), jnp.int32)] * _NBUF,
            [pltpu.VMEM((C, H), jnp.float32)] * _NBUF,
            pltpu.VMEM_SHARED((n_pad, H), jnp.float32),
            [pltpu.SemaphoreType.DMA] * _NBUF,
            [pltpu.SemaphoreType.DMA] * _NBUF,
        ],
    )
    def scatter_add(mij_hbm, row_hbm, out_hbm, idxs, vals, acc,
                    vsems, ssems):
        cid = lax.axis_index("c")
        sid = lax.axis_index("s")
        wid = sid * _NC + cid
        base = wid * epw

        zero = jnp.zeros((_L,), jnp.float32)

        def zrow(r, c2):
            for j in range(H // _L):
                vals[0][r, pl.ds(j * _L, _L)] = zero
            return c2

        lax.fori_loop(0, C, zrow, 0)
        nz = (rpt + C - 1) // C

        def zcopy(t, c2):
            off = jnp.minimum(t * C, rpt - C)
            pltpu.sync_copy(vals[0], acc.at[pl.ds(sid * rpt + off, C)])
            return c2

        lax.fori_loop(0, nz, zcopy, 0)
        plsc.subcore_barrier()

        def issue_copies(k, b):
            off = base + k * C
            pltpu.async_copy(row_hbm.at[pl.ds(off, C)], idxs[b], vsems[b])
            pltpu.async_copy(mij_hbm.at[pl.ds(off, C)], vals[b], vsems[b])

        def wait_copies(k, b):
            off = base + k * C
            pltpu.make_async_copy(row_hbm.at[pl.ds(off, C)], idxs[b],
                                  vsems[b]).wait()
            pltpu.make_async_copy(mij_hbm.at[pl.ds(off, C)], vals[b],
                                  vsems[b]).wait()

        def issue_scatter(b):
            pltpu.async_copy(vals[b], acc.at[idxs[b]], ssems[b], add=True)

        def wait_scatter(b):
            pltpu.make_async_copy(vals[b], acc.at[idxs[b]],
                                  ssems[b]).wait()

        issue_copies(0, 0)
        issue_copies(1, 1)
        wait_copies(0, 0)
        issue_scatter(0)

        def group(i, c):
            for p in range(_NBUF):
                k = _NBUF * i + 1 + p
                b = (p + 1) % _NBUF
                nk = k + 1
                nb = (b + 1) % _NBUF
                if p == 2:
                    wait_scatter(nb)
                    issue_copies(nk, nb)
                elif p == 0 or p == 1:
                    @pl.when(i > 0)
                    def _():
                        wait_scatter(nb)
                    issue_copies(nk, nb)
                else:
                    @pl.when(i < ngrp - 1)
                    def _():
                        wait_scatter(nb)
                        issue_copies(nk, nb)
                wait_copies(k, b)
                issue_scatter(b)
            return c

        lax.fori_loop(0, ngrp, group, 0)
        for b in range(_NBUF):
            wait_scatter(b)
        plsc.subcore_barrier()

        pltpu.sync_copy(acc.at[pl.ds(sid * rpt, rpt)],
                        out_hbm.at[cid, pl.ds(sid * rpt, rpt)])

    return scatter_add


def _node_body(hq_ref, p_ref, wq_ref, wa_ref, bn1_ref, wn2_ref, bn2_ref,
               out_ref):
    agg = p_ref[0] + p_ref[1]
    hq = hq_ref[...]
    t = _silu(jnp.dot(hq, wq_ref[...], preferred_element_type=jnp.float32)
              + jnp.dot(agg, wa_ref[...], preferred_element_type=jnp.float32)
              + bn1_ref[...])
    out_ref[...] = hq + jnp.dot(t, wn2_ref[...],
                                preferred_element_type=jnp.float32) + bn2_ref[...]


def kernel(h_q, h_kv, edge_index, We1, be1, We2, be2, Wn1, bn1, Wn2, bn2):
    N, D = h_q.shape
    H = We2.shape[0]
    E = edge_index.shape[1]
    row = edge_index[0]
    col = edge_index[1]
    W1q, W1k = We1[:D], We1[D:]
    Wn1q, Wn1a = Wn1[:D], Wn1[D:]

    BN = 400
    grid_n = N // BN
    aq, ak = pl.pallas_call(
        _proj_body,
        grid=(grid_n,),
        in_specs=[
            pl.BlockSpec((BN, D), lambda i: (i, 0)),
            pl.BlockSpec((BN, D), lambda i: (i, 0)),
            pl.BlockSpec((D, H), lambda i: (0, 0)),
            pl.BlockSpec((D, H), lambda i: (0, 0)),
            pl.BlockSpec((H,), lambda i: (0,)),
        ],
        out_specs=[
            pl.BlockSpec((BN, H), lambda i: (i, 0)),
            pl.BlockSpec((BN, H), lambda i: (i, 0)),
        ],
        out_shape=[jax.ShapeDtypeStruct((N, H), jnp.float32)] * 2,
    )(h_q, h_kv, W1q, W1k, be1)

    g = _make_gather_add(N, H, E)(aq, ak, row, col)

    BE = 2000
    mij = pl.pallas_call(
        _edge_body,
        grid=(E // BE,),
        in_specs=[
            pl.BlockSpec((BE, H), lambda i: (i, 0)),
            pl.BlockSpec((H, H), lambda i: (0, 0)),
            pl.BlockSpec((H,), lambda i: (0,)),
        ],
        out_specs=pl.BlockSpec((BE, H), lambda i: (i, 0)),
        out_shape=jax.ShapeDtypeStruct((E, H), jnp.float32),
    )(g, We2, be2)

    partials = _make_scatter_add(N, H, E)(mij, row)

    h_new = pl.pallas_call(
        _node_body,
        grid=(grid_n,),
        in_specs=[
            pl.BlockSpec((BN, D), lambda i: (i, 0)),
            pl.BlockSpec((_NC, BN, H), lambda i: (0, i, 0)),
            pl.BlockSpec((D, H), lambda i: (0, 0)),
            pl.BlockSpec((H, H), lambda i: (0, 0)),
            pl.BlockSpec((H,), lambda i: (0,)),
            pl.BlockSpec((H, D), lambda i: (0, 0)),
            pl.BlockSpec((D,), lambda i: (0,)),
        ],
        out_specs=pl.BlockSpec((BN, D), lambda i: (i, 0)),
        out_shape=jax.ShapeDtypeStruct((N, D), jnp.float32),
    )(h_q, partials, Wn1q, Wn1a, bn1, Wn2, bn2)

    return (h_new, mij)


# speedup vs baseline: 1.3595x; 1.3595x over previous
import functools

import jax
import jax.numpy as jnp
from jax import lax
from jax.experimental import pallas as pl
from jax.experimental.pallas import tpu as pltpu
from jax.experimental.pallas import tpu_sc as plsc

_NC = 2
_NS = 16
_NW = _NC * _NS
_L = 16


def _silu(x):
    return x * lax.logistic(x)


def _proj_body(hq_ref, hkv_ref, wq_ref, wk_ref, be1_ref, aq_ref, ak_ref):
    aq_ref[...] = jnp.dot(hq_ref[...], wq_ref[...],
                          preferred_element_type=jnp.float32)
    ak_ref[...] = jnp.dot(hkv_ref[...], wk_ref[...],
                          preferred_element_type=jnp.float32) + be1_ref[...]


_NBUF = 4
_GBUF = 5
_GLEAD = 3


@functools.lru_cache(maxsize=None)
def _make_gather_add(N, H, E):
    epw = E // _NW
    C = 80
    nch = epw // C
    ngrp = nch // _GBUF
    assert nch == ngrp * _GBUF
    mesh = plsc.VectorSubcoreMesh(core_axis_name="c", subcore_axis_name="s")

    @functools.partial(
        pl.kernel,
        out_type=jax.ShapeDtypeStruct((E, H), jnp.float32),
        mesh=mesh,
        scratch_types=[
            pltpu.VMEM((epw,), jnp.int32),
            pltpu.VMEM((epw,), jnp.int32),
            [pltpu.VMEM((C, H), jnp.float32)] * _GBUF,
            [pltpu.VMEM((C, H), jnp.float32)] * _GBUF,
            [pltpu.SemaphoreType.DMA] * _GBUF,
            [pltpu.SemaphoreType.DMA] * _GBUF,
        ],
    )
    def gather_add(aq_hbm, ak_hbm, row_hbm, col_hbm, out_hbm,
                   ridx, cidx, bqs, bks, gsems, osems):
        wid = lax.axis_index("s") * _NC + lax.axis_index("c")
        base = wid * epw

        pltpu.sync_copy(row_hbm.at[pl.ds(base, epw)], ridx)
        pltpu.sync_copy(col_hbm.at[pl.ds(base, epw)], cidx)

        def issue_gather(k, b):
            pltpu.async_copy(aq_hbm.at[ridx.at[pl.ds(k * C, C)]], bqs[b],
                             gsems[b])
            pltpu.async_copy(ak_hbm.at[cidx.at[pl.ds(k * C, C)]], bks[b],
                             gsems[b])

        def wait_gather(k, b):
            pltpu.make_async_copy(aq_hbm.at[ridx.at[pl.ds(k * C, C)]],
                                  bqs[b], gsems[b]).wait()
            pltpu.make_async_copy(ak_hbm.at[cidx.at[pl.ds(k * C, C)]],
                                  bks[b], gsems[b]).wait()

        def add_and_out(k, b):
            bq, bk = bqs[b], bks[b]

            def add_row(e, c2):
                for j in range(H // _L):
                    sl = pl.ds(j * _L, _L)
                    plsc.addupdate(bq.at[e, sl], bk[e, sl])
                return c2

            lax.fori_loop(0, C, add_row, 0, unroll=2)
            pltpu.async_copy(bq, out_hbm.at[pl.ds(base + k * C, C)],
                             osems[b])

        def wait_out(b):
            pltpu.make_async_copy(bqs[b], out_hbm.at[pl.ds(base, C)],
                                  osems[b]).wait()

        for k0 in range(_GLEAD):
            issue_gather(k0, k0)

        def group(i, c):
            for p in range(_GBUF):
                k = _GBUF * i + p
                nk = k + _GLEAD
                nb = (p + _GLEAD) % _GBUF
                if p < _GLEAD - 1:
                    @pl.when(i > 0)
                    def _():
                        wait_out(nb)
                    issue_gather(nk, nb)
                else:
                    @pl.when(i < ngrp - 1)
                    def _():
                        wait_out(nb)
                        issue_gather(nk, nb)
                wait_gather(k, p)
                add_and_out(k, p)
            return c

        lax.fori_loop(0, ngrp, group, 0)
        for b in range(_GBUF):
            wait_out(b)

    return gather_add


def _edge_body(g_ref, w2_ref, b2_ref, out_ref):
    h1 = _silu(g_ref[...])
    out_ref[...] = _silu(jnp.dot(h1, w2_ref[...],
                                 preferred_element_type=jnp.float32)
                         + b2_ref[...])


@functools.lru_cache(maxsize=None)
def _make_scatter_add(N, H, E):
    epw = E // _NW
    C = 80
    nch = epw // C
    n_pad = ((N + _NS * 8 - 1) // (_NS * 8)) * (_NS * 8)
    rpt = n_pad // _NS
    mesh = plsc.VectorSubcoreMesh(core_axis_name="c", subcore_axis_name="s")

    ngrp = (nch - 1) // _NBUF
    assert nch == 1 + ngrp * _NBUF

    @functools.partial(
        pl.kernel,
        out_type=jax.ShapeDtypeStruct((_NC, n_pad, H), jnp.float32),
        mesh=mesh,
        scratch_types=[
            [pltpu.VMEM((C,), jnp.int32)] * _NBUF,
            [pltpu.VMEM((C, H), jnp.float32)] * _NBUF,
            pltpu.VMEM_SHARED((n_pad, H), jnp.float32),
            [pltpu.SemaphoreType.DMA] * _NBUF,
            [pltpu.SemaphoreType.DMA] * _NBUF,
        ],
    )
    def scatter_add(mij_hbm, row_hbm, out_hbm, idxs, vals, acc,
                    vsems, ssems):
        cid = lax.axis_index("c")
        sid = lax.axis_index("s")
        wid = sid * _NC + cid
        base = wid * epw

        zero = jnp.zeros((_L,), jnp.float32)

        def zrow(r, c2):
            for j in range(H // _L):
                vals[0][r, pl.ds(j * _L, _L)] = zero
            return c2

        lax.fori_loop(0, C, zrow, 0)
        nz = (rpt + C - 1) // C

        def zcopy(t, c2):
            off = jnp.minimum(t * C, rpt - C)
            pltpu.sync_copy(vals[0], acc.at[pl.ds(sid * rpt + off, C)])
            return c2

        lax.fori_loop(0, nz, zcopy, 0)
        plsc.subcore_barrier()

        def issue_copies(k, b):
            off = base + k * C
            pltpu.async_copy(row_hbm.at[pl.ds(off, C)], idxs[b], vsems[b])
            pltpu.async_copy(mij_hbm.at[pl.ds(off, C)], vals[b], vsems[b])

        def wait_copies(k, b):
            off = base + k * C
            pltpu.make_async_copy(row_hbm.at[pl.ds(off, C)], idxs[b],
                                  vsems[b]).wait()
            pltpu.make_async_copy(mij_hbm.at[pl.ds(off, C)], vals[b],
                                  vsems[b]).wait()

        def issue_scatter(b):
            pltpu.async_copy(vals[b], acc.at[idxs[b]], ssems[b], add=True)

        def wait_scatter(b):
            pltpu.make_async_copy(vals[b], acc.at[idxs[b]],
                                  ssems[b]).wait()

        issue_copies(0, 0)
        issue_copies(1, 1)
        wait_copies(0, 0)
        issue_scatter(0)

        def group(i, c):
            for p in range(_NBUF):
                k = _NBUF * i + 1 + p
                b = (p + 1) % _NBUF
                nk = k + 1
                nb = (b + 1) % _NBUF
                if p == 2:
                    wait_scatter(nb)
                    issue_copies(nk, nb)
                elif p == 0 or p == 1:
                    @pl.when(i > 0)
                    def _():
                        wait_scatter(nb)
                    issue_copies(nk, nb)
                else:
                    @pl.when(i < ngrp - 1)
                    def _():
                        wait_scatter(nb)
                        issue_copies(nk, nb)
                wait_copies(k, b)
                issue_scatter(b)
            return c

        lax.fori_loop(0, ngrp, group, 0)
        for b in range(_NBUF):
            wait_scatter(b)
        plsc.subcore_barrier()

        pltpu.sync_copy(acc.at[pl.ds(sid * rpt, rpt)],
                        out_hbm.at[cid, pl.ds(sid * rpt, rpt)])

    return scatter_add


def _node_body(hq_ref, p_ref, wq_ref, wa_ref, bn1_ref, wn2_ref, bn2_ref,
               out_ref):
    agg = p_ref[0] + p_ref[1]
    hq = hq_ref[...]
    t = _silu(jnp.dot(hq, wq_ref[...], preferred_element_type=jnp.float32)
              + jnp.dot(agg, wa_ref[...], preferred_element_type=jnp.float32)
              + bn1_ref[...])
    out_ref[...] = hq + jnp.dot(t, wn2_ref[...],
                                preferred_element_type=jnp.float32) + bn2_ref[...]


def kernel(h_q, h_kv, edge_index, We1, be1, We2, be2, Wn1, bn1, Wn2, bn2):
    N, D = h_q.shape
    H = We2.shape[0]
    E = edge_index.shape[1]
    row = edge_index[0]
    col = edge_index[1]
    W1q, W1k = We1[:D], We1[D:]
    Wn1q, Wn1a = Wn1[:D], Wn1[D:]

    BN = 400
    grid_n = N // BN
    aq, ak = pl.pallas_call(
        _proj_body,
        grid=(grid_n,),
        in_specs=[
            pl.BlockSpec((BN, D), lambda i: (i, 0)),
            pl.BlockSpec((BN, D), lambda i: (i, 0)),
            pl.BlockSpec((D, H), lambda i: (0, 0)),
            pl.BlockSpec((D, H), lambda i: (0, 0)),
            pl.BlockSpec((H,), lambda i: (0,)),
        ],
        out_specs=[
            pl.BlockSpec((BN, H), lambda i: (i, 0)),
            pl.BlockSpec((BN, H), lambda i: (i, 0)),
        ],
        out_shape=[jax.ShapeDtypeStruct((N, H), jnp.float32)] * 2,
    )(h_q, h_kv, W1q, W1k, be1)

    g = _make_gather_add(N, H, E)(aq, ak, row, col)

    BE = 2000
    mij = pl.pallas_call(
        _edge_body,
        grid=(E // BE,),
        in_specs=[
            pl.BlockSpec((BE, H), lambda i: (i, 0)),
            pl.BlockSpec((H, H), lambda i: (0, 0)),
            pl.BlockSpec((H,), lambda i: (0,)),
        ],
        out_specs=pl.BlockSpec((BE, H), lambda i: (i, 0)),
        out_shape=jax.ShapeDtypeStruct((E, H), jnp.float32),
    )(g, We2, be2)

    partials = _make_scatter_add(N, H, E)(mij, row)

    h_new = pl.pallas_call(
        _node_body,
        grid=(grid_n,),
        in_specs=[
            pl.BlockSpec((BN, D), lambda i: (i, 0)),
            pl.BlockSpec((_NC, BN, H), lambda i: (0, i, 0)),
            pl.BlockSpec((D, H), lambda i: (0, 0)),
            pl.BlockSpec((H, H), lambda i: (0, 0)),
            pl.BlockSpec((H,), lambda i: (0,)),
            pl.BlockSpec((H, D), lambda i: (0, 0)),
            pl.BlockSpec((D,), lambda i: (0,)),
        ],
        out_specs=pl.BlockSpec((BN, D), lambda i: (i, 0)),
        out_shape=jax.ShapeDtypeStruct((N, D), jnp.float32),
    )(h_q, partials, Wn1q, Wn1a, bn1, Wn2, bn2)

    return (h_new, mij)


# speedup vs baseline: 1.5460x; 1.1372x over previous
import functools

import jax
import jax.numpy as jnp
from jax import lax
from jax.experimental import pallas as pl
from jax.experimental.pallas import tpu as pltpu
from jax.experimental.pallas import tpu_sc as plsc

_NC = 2
_NS = 16
_NW = _NC * _NS
_L = 16


def _silu(x):
    return x * lax.logistic(x)


def _proj_body(hq_ref, hkv_ref, w1_ref, be1_ref, aq_ref, ak_ref):
    d = hq_ref.shape[1]
    aq_ref[...] = jnp.dot(hq_ref[...], w1_ref[:d],
                          preferred_element_type=jnp.float32)
    ak_ref[...] = jnp.dot(hkv_ref[...], w1_ref[d:],
                          preferred_element_type=jnp.float32) + be1_ref[...]


_NBUF = 4
_GBUF = 5
_GLEAD = 3


@functools.lru_cache(maxsize=None)
def _make_gather_add(N, H, E):
    epw = E // _NW
    C = 80
    nch = epw // C
    ngrp = nch // _GBUF
    assert nch == ngrp * _GBUF
    mesh = plsc.VectorSubcoreMesh(core_axis_name="c", subcore_axis_name="s")

    @functools.partial(
        pl.kernel,
        out_type=jax.ShapeDtypeStruct((E, H), jnp.float32),
        mesh=mesh,
        scratch_types=[
            pltpu.VMEM((epw,), jnp.int32),
            pltpu.VMEM((epw,), jnp.int32),
            [pltpu.VMEM((C, H), jnp.float32)] * _GBUF,
            [pltpu.VMEM((C, H), jnp.float32)] * _GBUF,
            [pltpu.SemaphoreType.DMA] * _GBUF,
            [pltpu.SemaphoreType.DMA] * _GBUF,
        ],
    )
    def gather_add(aq_hbm, ak_hbm, row_hbm, col_hbm, out_hbm,
                   ridx, cidx, bqs, bks, gsems, osems):
        wid = lax.axis_index("s") * _NC + lax.axis_index("c")
        base = wid * epw

        pltpu.sync_copy(row_hbm.at[pl.ds(base, epw)], ridx)
        pltpu.sync_copy(col_hbm.at[pl.ds(base, epw)], cidx)

        def issue_gather(k, b):
            pltpu.async_copy(aq_hbm.at[ridx.at[pl.ds(k * C, C)]], bqs[b],
                             gsems[b])
            pltpu.async_copy(ak_hbm.at[cidx.at[pl.ds(k * C, C)]], bks[b],
                             gsems[b])

        def wait_gather(k, b):
            pltpu.make_async_copy(aq_hbm.at[ridx.at[pl.ds(k * C, C)]],
                                  bqs[b], gsems[b]).wait()
            pltpu.make_async_copy(ak_hbm.at[cidx.at[pl.ds(k * C, C)]],
                                  bks[b], gsems[b]).wait()

        def add_and_out(k, b):
            bq, bk = bqs[b], bks[b]

            def add_row(e, c2):
                for j in range(H // _L):
                    sl = pl.ds(j * _L, _L)
                    plsc.addupdate(bq.at[e, sl], bk[e, sl])
                return c2

            lax.fori_loop(0, C, add_row, 0, unroll=2)
            pltpu.async_copy(bq, out_hbm.at[pl.ds(base + k * C, C)],
                             osems[b])

        def wait_out(b):
            pltpu.make_async_copy(bqs[b], out_hbm.at[pl.ds(base, C)],
                                  osems[b]).wait()

        for k0 in range(_GLEAD):
            issue_gather(k0, k0)

        def group(i, c):
            for p in range(_GBUF):
                k = _GBUF * i + p
                nk = k + _GLEAD
                nb = (p + _GLEAD) % _GBUF
                if p < _GLEAD - 1:
                    @pl.when(i > 0)
                    def _():
                        wait_out(nb)
                    issue_gather(nk, nb)
                else:
                    @pl.when(i < ngrp - 1)
                    def _():
                        wait_out(nb)
                        issue_gather(nk, nb)
                wait_gather(k, p)
                add_and_out(k, p)
            return c

        lax.fori_loop(0, ngrp, group, 0)
        for b in range(_GBUF):
            wait_out(b)

    return gather_add


def _edge_body(g_ref, w2_ref, b2_ref, out_ref):
    h1 = _silu(g_ref[...])
    out_ref[...] = _silu(jnp.dot(h1, w2_ref[...],
                                 preferred_element_type=jnp.float32)
                         + b2_ref[...])


@functools.lru_cache(maxsize=None)
def _make_scatter_add(N, H, E):
    epw = E // _NW
    C = 80
    nch = epw // C
    n_pad = ((N + _NS * 8 - 1) // (_NS * 8)) * (_NS * 8)
    rpt = n_pad // _NS
    mesh = plsc.VectorSubcoreMesh(core_axis_name="c", subcore_axis_name="s")

    ngrp = (nch - 1) // _NBUF
    assert nch == 1 + ngrp * _NBUF

    @functools.partial(
        pl.kernel,
        out_type=jax.ShapeDtypeStruct((_NC, n_pad, H), jnp.float32),
        mesh=mesh,
        scratch_types=[
            [pltpu.VMEM((C,), jnp.int32)] * _NBUF,
            [pltpu.VMEM((C, H), jnp.float32)] * _NBUF,
            pltpu.VMEM_SHARED((n_pad, H), jnp.float32),
            [pltpu.SemaphoreType.DMA] * _NBUF,
            [pltpu.SemaphoreType.DMA] * _NBUF,
        ],
    )
    def scatter_add(mij_hbm, row_hbm, out_hbm, idxs, vals, acc,
                    vsems, ssems):
        cid = lax.axis_index("c")
        sid = lax.axis_index("s")
        wid = sid * _NC + cid
        base = wid * epw

        zero = jnp.zeros((_L,), jnp.float32)

        def zrow(r, c2):
            for j in range(H // _L):
                vals[0][r, pl.ds(j * _L, _L)] = zero
            return c2

        lax.fori_loop(0, C, zrow, 0)
        nz = (rpt + C - 1) // C

        def zcopy(t, c2):
            off = jnp.minimum(t * C, rpt - C)
            pltpu.sync_copy(vals[0], acc.at[pl.ds(sid * rpt + off, C)])
            return c2

        lax.fori_loop(0, nz, zcopy, 0)
        plsc.subcore_barrier()

        def issue_copies(k, b):
            off = base + k * C
            pltpu.async_copy(row_hbm.at[pl.ds(off, C)], idxs[b], vsems[b])
            pltpu.async_copy(mij_hbm.at[pl.ds(off, C)], vals[b], vsems[b])

        def wait_copies(k, b):
            off = base + k * C
            pltpu.make_async_copy(row_hbm.at[pl.ds(off, C)], idxs[b],
                                  vsems[b]).wait()
            pltpu.make_async_copy(mij_hbm.at[pl.ds(off, C)], vals[b],
                                  vsems[b]).wait()

        def issue_scatter(b):
            pltpu.async_copy(vals[b], acc.at[idxs[b]], ssems[b], add=True)

        def wait_scatter(b):
            pltpu.make_async_copy(vals[b], acc.at[idxs[b]],
                                  ssems[b]).wait()

        issue_copies(0, 0)
        issue_copies(1, 1)
        wait_copies(0, 0)
        issue_scatter(0)

        def group(i, c):
            for p in range(_NBUF):
                k = _NBUF * i + 1 + p
                b = (p + 1) % _NBUF
                nk = k + 1
                nb = (b + 1) % _NBUF
                if p == 2:
                    wait_scatter(nb)
                    issue_copies(nk, nb)
                elif p == 0 or p == 1:
                    @pl.when(i > 0)
                    def _():
                        wait_scatter(nb)
                    issue_copies(nk, nb)
                else:
                    @pl.when(i < ngrp - 1)
                    def _():
                        wait_scatter(nb)
                        issue_copies(nk, nb)
                wait_copies(k, b)
                issue_scatter(b)
            return c

        lax.fori_loop(0, ngrp, group, 0)
        for b in range(_NBUF):
            wait_scatter(b)
        plsc.subcore_barrier()

        pltpu.sync_copy(acc.at[pl.ds(sid * rpt, rpt)],
                        out_hbm.at[cid, pl.ds(sid * rpt, rpt)])

    return scatter_add


def _node_body(hq_ref, p_ref, wn1_ref, bn1_ref, wn2_ref, bn2_ref,
               out_ref):
    agg = p_ref[0] + p_ref[1]
    hq = hq_ref[...]
    d = hq_ref.shape[1]
    t = _silu(jnp.dot(hq, wn1_ref[:d], preferred_element_type=jnp.float32)
              + jnp.dot(agg, wn1_ref[d:], preferred_element_type=jnp.float32)
              + bn1_ref[...])
    out_ref[...] = hq + jnp.dot(t, wn2_ref[...],
                                preferred_element_type=jnp.float32) + bn2_ref[...]


def kernel(h_q, h_kv, edge_index, We1, be1, We2, be2, Wn1, bn1, Wn2, bn2):
    N, D = h_q.shape
    H = We2.shape[0]
    E = edge_index.shape[1]
    row = edge_index[0]
    col = edge_index[1]

    BN = 2000
    grid_n = N // BN
    aq, ak = pl.pallas_call(
        _proj_body,
        grid=(grid_n,),
        in_specs=[
            pl.BlockSpec((BN, D), lambda i: (i, 0)),
            pl.BlockSpec((BN, D), lambda i: (i, 0)),
            pl.BlockSpec((2 * D, H), lambda i: (0, 0)),
            pl.BlockSpec((H,), lambda i: (0,)),
        ],
        out_specs=[
            pl.BlockSpec((BN, H), lambda i: (i, 0)),
            pl.BlockSpec((BN, H), lambda i: (i, 0)),
        ],
        out_shape=[jax.ShapeDtypeStruct((N, H), jnp.float32)] * 2,
    )(h_q, h_kv, We1, be1)

    g = _make_gather_add(N, H, E)(aq, ak, row, col)

    BE = 4000
    mij = pl.pallas_call(
        _edge_body,
        grid=(E // BE,),
        in_specs=[
            pl.BlockSpec((BE, H), lambda i: (i, 0)),
            pl.BlockSpec((H, H), lambda i: (0, 0)),
            pl.BlockSpec((H,), lambda i: (0,)),
        ],
        out_specs=pl.BlockSpec((BE, H), lambda i: (i, 0)),
        out_shape=jax.ShapeDtypeStruct((E, H), jnp.float32),
    )(g, We2, be2)

    partials = _make_scatter_add(N, H, E)(mij, row)

    h_new = pl.pallas_call(
        _node_body,
        grid=(grid_n,),
        in_specs=[
            pl.BlockSpec((BN, D), lambda i: (i, 0)),
            pl.BlockSpec((_NC, BN, H), lambda i: (0, i, 0)),
            pl.BlockSpec((2 * D, H), lambda i: (0, 0)),
            pl.BlockSpec((H,), lambda i: (0,)),
            pl.BlockSpec((H, D), lambda i: (0, 0)),
            pl.BlockSpec((D,), lambda i: (0,)),
        ],
        out_specs=pl.BlockSpec((BN, D), lambda i: (i, 0)),
        out_shape=jax.ShapeDtypeStruct((N, D), jnp.float32),
    )(h_q, partials, Wn1, bn1, Wn2, bn2)

    return (h_new, mij)


# speedup vs baseline: 1.6076x; 1.0398x over previous
import functools

import jax
import jax.numpy as jnp
from jax import lax
from jax.experimental import pallas as pl
from jax.experimental.pallas import tpu as pltpu
from jax.experimental.pallas import tpu_sc as plsc

_NC = 2
_NS = 16
_NW = _NC * _NS
_L = 16


def _silu(x):
    return x * lax.logistic(x)


def _proj_body(hq_ref, hkv_ref, w1_ref, be1_ref, aq_ref, ak_ref):
    d = hq_ref.shape[1]
    aq_ref[...] = jnp.dot(hq_ref[...], w1_ref[:d],
                          preferred_element_type=jnp.float32)
    ak_ref[...] = jnp.dot(hkv_ref[...], w1_ref[d:],
                          preferred_element_type=jnp.float32) + be1_ref[...]


_NBUF = 4
_GBUF = 5
_GLEAD = 3


@functools.lru_cache(maxsize=None)
def _make_gather_add(N, H, esz, eoff, C):
    epw = esz // _NW
    nch = epw // C
    ngrp = nch // _GBUF
    assert nch == ngrp * _GBUF
    mesh = plsc.VectorSubcoreMesh(core_axis_name="c", subcore_axis_name="s")

    @functools.partial(
        pl.kernel,
        out_type=jax.ShapeDtypeStruct((esz, H), jnp.float32),
        mesh=mesh,
        scratch_types=[
            pltpu.VMEM((epw,), jnp.int32),
            pltpu.VMEM((epw,), jnp.int32),
            [pltpu.VMEM((C, H), jnp.float32)] * _GBUF,
            [pltpu.VMEM((C, H), jnp.float32)] * _GBUF,
            [pltpu.SemaphoreType.DMA] * _GBUF,
            [pltpu.SemaphoreType.DMA] * _GBUF,
        ],
    )
    def gather_add(aq_hbm, ak_hbm, row_hbm, col_hbm, out_hbm,
                   ridx, cidx, bqs, bks, gsems, osems):
        wid = lax.axis_index("s") * _NC + lax.axis_index("c")
        base = wid * epw

        pltpu.sync_copy(row_hbm.at[pl.ds(eoff + base, epw)], ridx)
        pltpu.sync_copy(col_hbm.at[pl.ds(eoff + base, epw)], cidx)

        def issue_gather(k, b):
            pltpu.async_copy(aq_hbm.at[ridx.at[pl.ds(k * C, C)]], bqs[b],
                             gsems[b])
            pltpu.async_copy(ak_hbm.at[cidx.at[pl.ds(k * C, C)]], bks[b],
                             gsems[b])

        def wait_gather(k, b):
            pltpu.make_async_copy(aq_hbm.at[ridx.at[pl.ds(k * C, C)]],
                                  bqs[b], gsems[b]).wait()
            pltpu.make_async_copy(ak_hbm.at[cidx.at[pl.ds(k * C, C)]],
                                  bks[b], gsems[b]).wait()

        def add_and_out(k, b):
            bq, bk = bqs[b], bks[b]

            def add_row(e, c2):
                for j in range(H // _L):
                    sl = pl.ds(j * _L, _L)
                    plsc.addupdate(bq.at[e, sl], bk[e, sl])
                return c2

            lax.fori_loop(0, C, add_row, 0, unroll=2)
            pltpu.async_copy(bq, out_hbm.at[pl.ds(base + k * C, C)],
                             osems[b])

        def wait_out(b):
            pltpu.make_async_copy(bqs[b], out_hbm.at[pl.ds(base, C)],
                                  osems[b]).wait()

        for k0 in range(_GLEAD):
            issue_gather(k0, k0)

        def group(i, c):
            for p in range(_GBUF):
                k = _GBUF * i + p
                nk = k + _GLEAD
                nb = (p + _GLEAD) % _GBUF
                if p < _GLEAD - 1:
                    @pl.when(i > 0)
                    def _():
                        wait_out(nb)
                    issue_gather(nk, nb)
                else:
                    @pl.when(i < ngrp - 1)
                    def _():
                        wait_out(nb)
                        issue_gather(nk, nb)
                wait_gather(k, p)
                add_and_out(k, p)
            return c

        lax.fori_loop(0, ngrp, group, 0)
        for b in range(_GBUF):
            wait_out(b)

    return gather_add


def _edge_body(g_ref, w2_ref, b2_ref, out_ref):
    h1 = _silu(g_ref[...])
    out_ref[...] = _silu(jnp.dot(h1, w2_ref[...],
                                 preferred_element_type=jnp.float32)
                         + b2_ref[...])


def _edge_body_acc(prev_ref, g_ref, w2_ref, b2_ref, out_ref):
    del prev_ref
    _edge_body(g_ref, w2_ref, b2_ref, out_ref)


@functools.lru_cache(maxsize=None)
def _make_scatter_add(N, H, E):
    epw = E // _NW
    C = 80
    nch = epw // C
    n_pad = ((N + _NS * 8 - 1) // (_NS * 8)) * (_NS * 8)
    rpt = n_pad // _NS
    mesh = plsc.VectorSubcoreMesh(core_axis_name="c", subcore_axis_name="s")

    ngrp = (nch - 1) // _NBUF
    assert nch == 1 + ngrp * _NBUF

    @functools.partial(
        pl.kernel,
        out_type=jax.ShapeDtypeStruct((_NC, n_pad, H), jnp.float32),
        mesh=mesh,
        scratch_types=[
            [pltpu.VMEM((C,), jnp.int32)] * _NBUF,
            [pltpu.VMEM((C, H), jnp.float32)] * _NBUF,
            pltpu.VMEM_SHARED((n_pad, H), jnp.float32),
            [pltpu.SemaphoreType.DMA] * _NBUF,
            [pltpu.SemaphoreType.DMA] * _NBUF,
        ],
    )
    def scatter_add(mij_hbm, row_hbm, out_hbm, idxs, vals, acc,
                    vsems, ssems):
        cid = lax.axis_index("c")
        sid = lax.axis_index("s")
        wid = sid * _NC + cid
        base = wid * epw

        zero = jnp.zeros((_L,), jnp.float32)

        def zrow(r, c2):
            for j in range(H // _L):
                vals[0][r, pl.ds(j * _L, _L)] = zero
            return c2

        lax.fori_loop(0, C, zrow, 0)
        nz = (rpt + C - 1) // C

        def zcopy(t, c2):
            off = jnp.minimum(t * C, rpt - C)
            pltpu.sync_copy(vals[0], acc.at[pl.ds(sid * rpt + off, C)])
            return c2

        lax.fori_loop(0, nz, zcopy, 0)
        plsc.subcore_barrier()

        def issue_copies(k, b):
            off = base + k * C
            pltpu.async_copy(row_hbm.at[pl.ds(off, C)], idxs[b], vsems[b])
            pltpu.async_copy(mij_hbm.at[pl.ds(off, C)], vals[b], vsems[b])

        def wait_copies(k, b):
            off = base + k * C
            pltpu.make_async_copy(row_hbm.at[pl.ds(off, C)], idxs[b],
                                  vsems[b]).wait()
            pltpu.make_async_copy(mij_hbm.at[pl.ds(off, C)], vals[b],
                                  vsems[b]).wait()

        def issue_scatter(b):
            pltpu.async_copy(vals[b], acc.at[idxs[b]], ssems[b], add=True)

        def wait_scatter(b):
            pltpu.make_async_copy(vals[b], acc.at[idxs[b]],
                                  ssems[b]).wait()

        issue_copies(0, 0)
        issue_copies(1, 1)
        wait_copies(0, 0)
        issue_scatter(0)

        def group(i, c):
            for p in range(_NBUF):
                k = _NBUF * i + 1 + p
                b = (p + 1) % _NBUF
                nk = k + 1
                nb = (b + 1) % _NBUF
                if p == 2:
                    wait_scatter(nb)
                    issue_copies(nk, nb)
                elif p == 0 or p == 1:
                    @pl.when(i > 0)
                    def _():
                        wait_scatter(nb)
                    issue_copies(nk, nb)
                else:
                    @pl.when(i < ngrp - 1)
                    def _():
                        wait_scatter(nb)
                        issue_copies(nk, nb)
                wait_copies(k, b)
                issue_scatter(b)
            return c

        lax.fori_loop(0, ngrp, group, 0)
        for b in range(_NBUF):
            wait_scatter(b)
        plsc.subcore_barrier()

        pltpu.sync_copy(acc.at[pl.ds(sid * rpt, rpt)],
                        out_hbm.at[cid, pl.ds(sid * rpt, rpt)])

    return scatter_add


def _node_body(hq_ref, p_ref, wn1_ref, bn1_ref, wn2_ref, bn2_ref,
               out_ref):
    agg = p_ref[0] + p_ref[1]
    hq = hq_ref[...]
    d = hq_ref.shape[1]
    t = _silu(jnp.dot(hq, wn1_ref[:d], preferred_element_type=jnp.float32)
              + jnp.dot(agg, wn1_ref[d:], preferred_element_type=jnp.float32)
              + bn1_ref[...])
    out_ref[...] = hq + jnp.dot(t, wn2_ref[...],
                                preferred_element_type=jnp.float32) + bn2_ref[...]


def kernel(h_q, h_kv, edge_index, We1, be1, We2, be2, Wn1, bn1, Wn2, bn2):
    N, D = h_q.shape
    H = We2.shape[0]
    E = edge_index.shape[1]
    row = edge_index[0]
    col = edge_index[1]

    BN = 2000
    grid_n = N // BN
    aq, ak = pl.pallas_call(
        _proj_body,
        grid=(grid_n,),
        in_specs=[
            pl.BlockSpec((BN, D), lambda i: (i, 0)),
            pl.BlockSpec((BN, D), lambda i: (i, 0)),
            pl.BlockSpec((2 * D, H), lambda i: (0, 0)),
            pl.BlockSpec((H,), lambda i: (0,)),
        ],
        out_specs=[
            pl.BlockSpec((BN, H), lambda i: (i, 0)),
            pl.BlockSpec((BN, H), lambda i: (i, 0)),
        ],
        out_shape=[jax.ShapeDtypeStruct((N, H), jnp.float32)] * 2,
    )(h_q, h_kv, We1, be1)

    E2 = E // 2
    g0 = _make_gather_add(N, H, E2, 0, 40)(aq, ak, row, col)
    g1 = _make_gather_add(N, H, E2, E2, 40)(aq, ak, row, col)

    BE = 4000
    nblk = E2 // BE
    mij0 = pl.pallas_call(
        _edge_body,
        grid=(nblk,),
        in_specs=[
            pl.BlockSpec((BE, H), lambda i: (i, 0)),
            pl.BlockSpec((H, H), lambda i: (0, 0)),
            pl.BlockSpec((H,), lambda i: (0,)),
        ],
        out_specs=pl.BlockSpec((BE, H), lambda i: (i, 0)),
        out_shape=jax.ShapeDtypeStruct((E, H), jnp.float32),
    )(g0, We2, be2)
    mij = pl.pallas_call(
        _edge_body_acc,
        grid=(nblk,),
        in_specs=[
            pl.BlockSpec(memory_space=pltpu.HBM),
            pl.BlockSpec((BE, H), lambda i: (i, 0)),
            pl.BlockSpec((H, H), lambda i: (0, 0)),
            pl.BlockSpec((H,), lambda i: (0,)),
        ],
        out_specs=pl.BlockSpec((BE, H), lambda i, _n=nblk: (i + _n, 0)),
        out_shape=jax.ShapeDtypeStruct((E, H), jnp.float32),
        input_output_aliases={0: 0},
    )(mij0, g1, We2, be2)

    partials = _make_scatter_add(N, H, E)(mij, row)

    h_new = pl.pallas_call(
        _node_body,
        grid=(grid_n,),
        in_specs=[
            pl.BlockSpec((BN, D), lambda i: (i, 0)),
            pl.BlockSpec((_NC, BN, H), lambda i: (0, i, 0)),
            pl.BlockSpec((2 * D, H), lambda i: (0, 0)),
            pl.BlockSpec((H,), lambda i: (0,)),
            pl.BlockSpec((H, D), lambda i: (0, 0)),
            pl.BlockSpec((D,), lambda i: (0,)),
        ],
        out_specs=pl.BlockSpec((BN, D), lambda i: (i, 0)),
        out_shape=jax.ShapeDtypeStruct((N, D), jnp.float32),
    )(h_q, partials, Wn1, bn1, Wn2, bn2)

    return (h_new, mij)


# speedup vs baseline: 1.6460x; 1.0239x over previous
import functools

import jax
import jax.numpy as jnp
from jax import lax
from jax.experimental import pallas as pl
from jax.experimental.pallas import tpu as pltpu
from jax.experimental.pallas import tpu_sc as plsc

_NC = 2
_NS = 16
_NW = _NC * _NS
_L = 16


def _silu(x):
    return x * lax.logistic(x)


def _split_body(e_ref, row_ref, col_ref):
    row_ref[...] = e_ref[0]
    col_ref[...] = e_ref[1]


def _proj_body(hq_ref, hkv_ref, w1_ref, be1_ref, aq_ref, ak_ref):
    d = hq_ref.shape[1]
    aq_ref[...] = jnp.dot(hq_ref[...], w1_ref[:d],
                          preferred_element_type=jnp.float32)
    ak_ref[...] = jnp.dot(hkv_ref[...], w1_ref[d:],
                          preferred_element_type=jnp.float32) + be1_ref[...]


_NBUF = 4
_GBUF = 5
_GLEAD = 3


@functools.lru_cache(maxsize=None)
def _make_gather_add(N, H, esz, eoff, C):
    epw = esz // _NW
    nch = epw // C
    ngrp = nch // _GBUF
    assert nch == ngrp * _GBUF
    mesh = plsc.VectorSubcoreMesh(core_axis_name="c", subcore_axis_name="s")

    @functools.partial(
        pl.kernel,
        out_type=jax.ShapeDtypeStruct((esz, H), jnp.float32),
        mesh=mesh,
        scratch_types=[
            pltpu.VMEM((epw,), jnp.int32),
            pltpu.VMEM((epw,), jnp.int32),
            [pltpu.VMEM((C, H), jnp.float32)] * _GBUF,
            [pltpu.VMEM((C, H), jnp.float32)] * _GBUF,
            [pltpu.SemaphoreType.DMA] * _GBUF,
            [pltpu.SemaphoreType.DMA] * _GBUF,
        ],
    )
    def gather_add(aq_hbm, ak_hbm, row_hbm, col_hbm, out_hbm,
                   ridx, cidx, bqs, bks, gsems, osems):
        wid = lax.axis_index("s") * _NC + lax.axis_index("c")
        base = wid * epw

        pltpu.sync_copy(row_hbm.at[pl.ds(eoff + base, epw)], ridx)
        pltpu.sync_copy(col_hbm.at[pl.ds(eoff + base, epw)], cidx)

        def issue_gather(k, b):
            pltpu.async_copy(aq_hbm.at[ridx.at[pl.ds(k * C, C)]], bqs[b],
                             gsems[b])
            pltpu.async_copy(ak_hbm.at[cidx.at[pl.ds(k * C, C)]], bks[b],
                             gsems[b])

        def wait_gather(k, b):
            pltpu.make_async_copy(aq_hbm.at[ridx.at[pl.ds(k * C, C)]],
                                  bqs[b], gsems[b]).wait()
            pltpu.make_async_copy(ak_hbm.at[cidx.at[pl.ds(k * C, C)]],
                                  bks[b], gsems[b]).wait()

        def add_and_out(k, b):
            bq, bk = bqs[b], bks[b]

            def add_row(e, c2):
                for j in range(H // _L):
                    sl = pl.ds(j * _L, _L)
                    plsc.addupdate(bq.at[e, sl], bk[e, sl])
                return c2

            lax.fori_loop(0, C, add_row, 0)
            pltpu.async_copy(bq, out_hbm.at[pl.ds(base + k * C, C)],
                             osems[b])

        def wait_out(b):
            pltpu.make_async_copy(bqs[b], out_hbm.at[pl.ds(base, C)],
                                  osems[b]).wait()

        for k0 in range(_GLEAD):
            issue_gather(k0, k0)

        def group(i, c):
            for p in range(_GBUF):
                k = _GBUF * i + p
                nk = k + _GLEAD
                nb = (p + _GLEAD) % _GBUF
                if p < _GLEAD - 1:
                    @pl.when(i > 0)
                    def _():
                        wait_out(nb)
                    issue_gather(nk, nb)
                else:
                    @pl.when(i < ngrp - 1)
                    def _():
                        wait_out(nb)
                        issue_gather(nk, nb)
                wait_gather(k, p)
                add_and_out(k, p)
            return c

        lax.fori_loop(0, ngrp, group, 0)
        for b in range(_GBUF):
            wait_out(b)

    return gather_add


def _edge_body(g_ref, w2_ref, b2_ref, out_ref):
    h1 = _silu(g_ref[...])
    out_ref[...] = _silu(jnp.dot(h1, w2_ref[...],
                                 preferred_element_type=jnp.float32)
                         + b2_ref[...])


def _edge_body_acc(prev_ref, g_ref, w2_ref, b2_ref, out_ref):
    del prev_ref
    _edge_body(g_ref, w2_ref, b2_ref, out_ref)


@functools.lru_cache(maxsize=None)
def _make_scatter_add(N, H, E):
    epw = E // _NW
    C = 80
    nch = epw // C
    n_pad = ((N + _NS * 8 - 1) // (_NS * 8)) * (_NS * 8)
    rpt = n_pad // _NS
    mesh = plsc.VectorSubcoreMesh(core_axis_name="c", subcore_axis_name="s")

    ngrp = (nch - 1) // _NBUF
    assert nch == 1 + ngrp * _NBUF

    @functools.partial(
        pl.kernel,
        out_type=jax.ShapeDtypeStruct((_NC, n_pad, H), jnp.float32),
        mesh=mesh,
        scratch_types=[
            [pltpu.VMEM((C,), jnp.int32)] * _NBUF,
            [pltpu.VMEM((C, H), jnp.float32)] * _NBUF,
            pltpu.VMEM_SHARED((n_pad, H), jnp.float32),
            [pltpu.SemaphoreType.DMA] * _NBUF,
            [pltpu.SemaphoreType.DMA] * _NBUF,
        ],
    )
    def scatter_add(mij_hbm, row_hbm, out_hbm, idxs, vals, acc,
                    vsems, ssems):
        cid = lax.axis_index("c")
        sid = lax.axis_index("s")
        wid = sid * _NC + cid
        base = wid * epw

        zero = jnp.zeros((_L,), jnp.float32)

        def zrow(r, c2):
            for j in range(H // _L):
                vals[0][r, pl.ds(j * _L, _L)] = zero
            return c2

        lax.fori_loop(0, C, zrow, 0)
        nz = (rpt + C - 1) // C

        def zcopy(t, c2):
            off = jnp.minimum(t * C, rpt - C)
            pltpu.sync_copy(vals[0], acc.at[pl.ds(sid * rpt + off, C)])
            return c2

        lax.fori_loop(0, nz, zcopy, 0)
        plsc.subcore_barrier()

        def issue_copies(k, b):
            off = base + k * C
            pltpu.async_copy(row_hbm.at[pl.ds(off, C)], idxs[b], vsems[b])
            pltpu.async_copy(mij_hbm.at[pl.ds(off, C)], vals[b], vsems[b])

        def wait_copies(k, b):
            off = base + k * C
            pltpu.make_async_copy(row_hbm.at[pl.ds(off, C)], idxs[b],
                                  vsems[b]).wait()
            pltpu.make_async_copy(mij_hbm.at[pl.ds(off, C)], vals[b],
                                  vsems[b]).wait()

        def issue_scatter(b):
            pltpu.async_copy(vals[b], acc.at[idxs[b]], ssems[b], add=True)

        def wait_scatter(b):
            pltpu.make_async_copy(vals[b], acc.at[idxs[b]],
                                  ssems[b]).wait()

        issue_copies(0, 0)
        issue_copies(1, 1)
        wait_copies(0, 0)
        issue_scatter(0)

        def group(i, c):
            for p in range(_NBUF):
                k = _NBUF * i + 1 + p
                b = (p + 1) % _NBUF
                nk = k + 1
                nb = (b + 1) % _NBUF
                if p == 2:
                    wait_scatter(nb)
                    issue_copies(nk, nb)
                elif p == 0 or p == 1:
                    @pl.when(i > 0)
                    def _():
                        wait_scatter(nb)
                    issue_copies(nk, nb)
                else:
                    @pl.when(i < ngrp - 1)
                    def _():
                        wait_scatter(nb)
                        issue_copies(nk, nb)
                wait_copies(k, b)
                issue_scatter(b)
            return c

        lax.fori_loop(0, ngrp, group, 0)
        for b in range(_NBUF):
            wait_scatter(b)
        plsc.subcore_barrier()

        pltpu.sync_copy(acc.at[pl.ds(sid * rpt, rpt)],
                        out_hbm.at[cid, pl.ds(sid * rpt, rpt)])

    return scatter_add


def _node_body(hq_ref, p_ref, wn1_ref, bn1_ref, wn2_ref, bn2_ref,
               out_ref):
    agg = p_ref[0] + p_ref[1]
    hq = hq_ref[...]
    d = hq_ref.shape[1]
    t = _silu(jnp.dot(hq, wn1_ref[:d], preferred_element_type=jnp.float32)
              + jnp.dot(agg, wn1_ref[d:], preferred_element_type=jnp.float32)
              + bn1_ref[...])
    out_ref[...] = hq + jnp.dot(t, wn2_ref[...],
                                preferred_element_type=jnp.float32) + bn2_ref[...]


def kernel(h_q, h_kv, edge_index, We1, be1, We2, be2, Wn1, bn1, Wn2, bn2):
    N, D = h_q.shape
    H = We2.shape[0]
    E = edge_index.shape[1]

    row, col = pl.pallas_call(
        _split_body,
        out_shape=[jax.ShapeDtypeStruct((E,), jnp.int32)] * 2,
    )(edge_index)

    BN = 2000
    grid_n = N // BN
    aq, ak = pl.pallas_call(
        _proj_body,
        grid=(grid_n,),
        in_specs=[
            pl.BlockSpec((BN, D), lambda i: (i, 0)),
            pl.BlockSpec((BN, D), lambda i: (i, 0)),
            pl.BlockSpec((2 * D, H), lambda i: (0, 0)),
            pl.BlockSpec((H,), lambda i: (0,)),
        ],
        out_specs=[
            pl.BlockSpec((BN, H), lambda i: (i, 0)),
            pl.BlockSpec((BN, H), lambda i: (i, 0)),
        ],
        out_shape=[jax.ShapeDtypeStruct((N, H), jnp.float32)] * 2,
    )(h_q, h_kv, We1, be1)

    E2 = E // 2
    g0 = _make_gather_add(N, H, E2, 0, 40)(aq, ak, row, col)
    g1 = _make_gather_add(N, H, E2, E2, 40)(aq, ak, row, col)

    BE = 4000
    nblk = E2 // BE
    mij0 = pl.pallas_call(
        _edge_body,
        grid=(nblk,),
        in_specs=[
            pl.BlockSpec((BE, H), lambda i: (i, 0)),
            pl.BlockSpec((H, H), lambda i: (0, 0)),
            pl.BlockSpec((H,), lambda i: (0,)),
        ],
        out_specs=pl.BlockSpec((BE, H), lambda i: (i, 0)),
        out_shape=jax.ShapeDtypeStruct((E, H), jnp.float32),
    )(g0, We2, be2)
    mij = pl.pallas_call(
        _edge_body_acc,
        grid=(nblk,),
        in_specs=[
            pl.BlockSpec(memory_space=pltpu.HBM),
            pl.BlockSpec((BE, H), lambda i: (i, 0)),
            pl.BlockSpec((H, H), lambda i: (0, 0)),
            pl.BlockSpec((H,), lambda i: (0,)),
        ],
        out_specs=pl.BlockSpec((BE, H), lambda i, _n=nblk: (i + _n, 0)),
        out_shape=jax.ShapeDtypeStruct((E, H), jnp.float32),
        input_output_aliases={0: 0},
    )(mij0, g1, We2, be2)

    partials = _make_scatter_add(N, H, E)(mij, row)

    h_new = pl.pallas_call(
        _node_body,
        grid=(grid_n,),
        in_specs=[
            pl.BlockSpec((BN, D), lambda i: (i, 0)),
            pl.BlockSpec((_NC, BN, H), lambda i: (0, i, 0)),
            pl.BlockSpec((2 * D, H), lambda i: (0, 0)),
            pl.BlockSpec((H,), lambda i: (0,)),
            pl.BlockSpec((H, D), lambda i: (0, 0)),
            pl.BlockSpec((D,), lambda i: (0,)),
        ],
        out_specs=pl.BlockSpec((BN, D), lambda i: (i, 0)),
        out_shape=jax.ShapeDtypeStruct((N, D), jnp.float32),
    )(h_q, partials, Wn1, bn1, Wn2, bn2)

    return (h_new, mij)


# speedup vs baseline: 1.6860x; 1.0243x over previous
import functools

import jax
import jax.numpy as jnp
from jax import lax
from jax.experimental import pallas as pl
from jax.experimental.pallas import tpu as pltpu
from jax.experimental.pallas import tpu_sc as plsc

_NC = 2
_NS = 16
_NW = _NC * _NS
_L = 16


def _silu(x):
    return x * lax.logistic(x)


def _split_body(e_ref, row_ref, col_ref):
    row_ref[...] = e_ref[0]
    col_ref[...] = e_ref[1]


def _proj_body(hq_ref, hkv_ref, w1_ref, be1_ref, aq_ref, ak_ref):
    d = hq_ref.shape[1]
    aq_ref[...] = jnp.dot(hq_ref[...], w1_ref[:d],
                          preferred_element_type=jnp.float32)
    ak_ref[...] = jnp.dot(hkv_ref[...], w1_ref[d:],
                          preferred_element_type=jnp.float32) + be1_ref[...]


_NBUF = 4
_GBUF = 5
_GLEAD = 3


@functools.lru_cache(maxsize=None)
def _make_gather_add(N, H, esz, eoff, C):
    epw = esz // _NW
    nch = epw // C
    ngrp = nch // _GBUF
    assert nch == ngrp * _GBUF
    mesh = plsc.VectorSubcoreMesh(core_axis_name="c", subcore_axis_name="s")

    @functools.partial(
        pl.kernel,
        out_type=jax.ShapeDtypeStruct((esz, H), jnp.float32),
        mesh=mesh,
        scratch_types=[
            pltpu.VMEM((epw,), jnp.int32),
            pltpu.VMEM((epw,), jnp.int32),
            [pltpu.VMEM((C, H), jnp.float32)] * _GBUF,
            [pltpu.VMEM((C, H), jnp.float32)] * _GBUF,
            [pltpu.SemaphoreType.DMA] * _GBUF,
            [pltpu.SemaphoreType.DMA] * _GBUF,
        ],
    )
    def gather_add(aq_hbm, ak_hbm, row_hbm, col_hbm, out_hbm,
                   ridx, cidx, bqs, bks, gsems, osems):
        wid = lax.axis_index("s") * _NC + lax.axis_index("c")
        base = wid * epw

        pltpu.sync_copy(row_hbm.at[pl.ds(eoff + base, epw)], ridx)
        pltpu.sync_copy(col_hbm.at[pl.ds(eoff + base, epw)], cidx)

        def issue_gather(k, b):
            pltpu.async_copy(aq_hbm.at[ridx.at[pl.ds(k * C, C)]], bqs[b],
                             gsems[b])
            pltpu.async_copy(ak_hbm.at[cidx.at[pl.ds(k * C, C)]], bks[b],
                             gsems[b])

        def wait_gather(k, b):
            pltpu.make_async_copy(aq_hbm.at[ridx.at[pl.ds(k * C, C)]],
                                  bqs[b], gsems[b]).wait()
            pltpu.make_async_copy(ak_hbm.at[cidx.at[pl.ds(k * C, C)]],
                                  bks[b], gsems[b]).wait()

        def add_and_out(k, b):
            bq, bk = bqs[b], bks[b]

            def add_row(e, c2):
                for j in range(H // _L):
                    sl = pl.ds(j * _L, _L)
                    plsc.addupdate(bq.at[e, sl], bk[e, sl])
                return c2

            lax.fori_loop(0, C, add_row, 0)
            pltpu.async_copy(bq, out_hbm.at[pl.ds(base + k * C, C)],
                             osems[b])

        def wait_out(b):
            pltpu.make_async_copy(bqs[b], out_hbm.at[pl.ds(base, C)],
                                  osems[b]).wait()

        for k0 in range(_GLEAD):
            issue_gather(k0, k0)

        def group(i, c):
            for p in range(_GBUF):
                k = _GBUF * i + p
                nk = k + _GLEAD
                nb = (p + _GLEAD) % _GBUF
                if p < _GLEAD - 1:
                    @pl.when(i > 0)
                    def _():
                        wait_out(nb)
                    issue_gather(nk, nb)
                else:
                    @pl.when(i < ngrp - 1)
                    def _():
                        wait_out(nb)
                        issue_gather(nk, nb)
                wait_gather(k, p)
                add_and_out(k, p)
            return c

        lax.fori_loop(0, ngrp, group, 0)
        for b in range(_GBUF):
            wait_out(b)

    return gather_add


def _edge_body(g_ref, w2_ref, b2_ref, out_ref):
    h1 = _silu(g_ref[...])
    out_ref[...] = _silu(jnp.dot(h1, w2_ref[...],
                                 preferred_element_type=jnp.float32)
                         + b2_ref[...])


def _edge_body_acc(prev_ref, g_ref, w2_ref, b2_ref, out_ref):
    del prev_ref
    _edge_body(g_ref, w2_ref, b2_ref, out_ref)


@functools.lru_cache(maxsize=None)
def _make_scatter_add(N, H, E):
    epw = E // _NW
    C = 80
    nch = epw // C
    n_pad = ((N + _NS * 8 - 1) // (_NS * 8)) * (_NS * 8)
    rpt = n_pad // _NS
    mesh = plsc.VectorSubcoreMesh(core_axis_name="c", subcore_axis_name="s")

    ngrp = (nch - 1) // _NBUF
    assert nch == 1 + ngrp * _NBUF

    @functools.partial(
        pl.kernel,
        out_type=jax.ShapeDtypeStruct((_NC, n_pad, H), jnp.float32),
        mesh=mesh,
        scratch_types=[
            [pltpu.VMEM((C,), jnp.int32)] * _NBUF,
            [pltpu.VMEM((C, H), jnp.float32)] * _NBUF,
            pltpu.VMEM_SHARED((n_pad, H), jnp.float32),
            [pltpu.SemaphoreType.DMA] * _NBUF,
            [pltpu.SemaphoreType.DMA] * _NBUF,
        ],
    )
    def scatter_add(mij_hbm, row_hbm, out_hbm, idxs, vals, acc,
                    vsems, ssems):
        cid = lax.axis_index("c")
        sid = lax.axis_index("s")
        wid = sid * _NC + cid
        base = wid * epw

        zero = jnp.zeros((_L,), jnp.float32)

        def zrow(r, c2):
            for j in range(H // _L):
                vals[0][r, pl.ds(j * _L, _L)] = zero
            return c2

        lax.fori_loop(0, C, zrow, 0)
        nz = (rpt + C - 1) // C

        def zcopy(t, c2):
            off = jnp.minimum(t * C, rpt - C)
            pltpu.sync_copy(vals[0], acc.at[pl.ds(sid * rpt + off, C)])
            return c2

        lax.fori_loop(0, nz, zcopy, 0)
        plsc.subcore_barrier()

        def issue_copies(k, b):
            off = base + k * C
            pltpu.async_copy(row_hbm.at[pl.ds(off, C)], idxs[b], vsems[b])
            pltpu.async_copy(mij_hbm.at[pl.ds(off, C)], vals[b], vsems[b])

        def wait_copies(k, b):
            off = base + k * C
            pltpu.make_async_copy(row_hbm.at[pl.ds(off, C)], idxs[b],
                                  vsems[b]).wait()
            pltpu.make_async_copy(mij_hbm.at[pl.ds(off, C)], vals[b],
                                  vsems[b]).wait()

        def issue_scatter(b):
            pltpu.async_copy(vals[b], acc.at[idxs[b]], ssems[b], add=True)

        def wait_scatter(b):
            pltpu.make_async_copy(vals[b], acc.at[idxs[b]],
                                  ssems[b]).wait()

        issue_copies(0, 0)
        issue_copies(1, 1)
        wait_copies(0, 0)
        issue_scatter(0)

        def group(i, c):
            for p in range(_NBUF):
                k = _NBUF * i + 1 + p
                b = (p + 1) % _NBUF
                nk = k + 1
                nb = (b + 1) % _NBUF
                if p == 2:
                    wait_scatter(nb)
                    issue_copies(nk, nb)
                elif p == 0 or p == 1:
                    @pl.when(i > 0)
                    def _():
                        wait_scatter(nb)
                    issue_copies(nk, nb)
                else:
                    @pl.when(i < ngrp - 1)
                    def _():
                        wait_scatter(nb)
                        issue_copies(nk, nb)
                wait_copies(k, b)
                issue_scatter(b)
            return c

        lax.fori_loop(0, ngrp, group, 0)
        for b in range(_NBUF):
            wait_scatter(b)
        plsc.subcore_barrier()

        pltpu.sync_copy(acc.at[pl.ds(sid * rpt, rpt)],
                        out_hbm.at[cid, pl.ds(sid * rpt, rpt)])

    return scatter_add


def _node_body(hq_ref, p_ref, wn1_ref, bn1_ref, wn2_ref, bn2_ref,
               out_ref):
    agg = p_ref[0] + p_ref[1]
    hq = hq_ref[...]
    d = hq_ref.shape[1]
    t = _silu(jnp.dot(hq, wn1_ref[:d], preferred_element_type=jnp.float32)
              + jnp.dot(agg, wn1_ref[d:], preferred_element_type=jnp.float32)
              + bn1_ref[...])
    out_ref[...] = hq + jnp.dot(t, wn2_ref[...],
                                preferred_element_type=jnp.float32) + bn2_ref[...]


def kernel(h_q, h_kv, edge_index, We1, be1, We2, be2, Wn1, bn1, Wn2, bn2):
    N, D = h_q.shape
    H = We2.shape[0]
    E = edge_index.shape[1]

    row, col = pl.pallas_call(
        _split_body,
        out_shape=[jax.ShapeDtypeStruct((E,), jnp.int32)] * 2,
    )(edge_index)

    BN = 2000
    grid_n = N // BN
    aq, ak = pl.pallas_call(
        _proj_body,
        grid=(grid_n,),
        in_specs=[
            pl.BlockSpec((BN, D), lambda i: (i, 0)),
            pl.BlockSpec((BN, D), lambda i: (i, 0)),
            pl.BlockSpec((2 * D, H), lambda i: (0, 0)),
            pl.BlockSpec((H,), lambda i: (0,)),
        ],
        out_specs=[
            pl.BlockSpec((BN, H), lambda i: (i, 0)),
            pl.BlockSpec((BN, H), lambda i: (i, 0)),
        ],
        out_shape=[jax.ShapeDtypeStruct((N, H), jnp.float32)] * 2,
    )(h_q, h_kv, We1, be1)

    E2 = E // 2
    g0 = _make_gather_add(N, H, E2, 0, 40)(aq, ak, row, col)
    g1 = _make_gather_add(N, H, E2, E2, 40)(aq, ak, row, col)

    BE = 8000
    nblk = E2 // BE
    mij0 = pl.pallas_call(
        _edge_body,
        grid=(nblk,),
        in_specs=[
            pl.BlockSpec((BE, H), lambda i: (i, 0)),
            pl.BlockSpec((H, H), lambda i: (0, 0)),
            pl.BlockSpec((H,), lambda i: (0,)),
        ],
        out_specs=pl.BlockSpec((BE, H), lambda i: (i, 0)),
        out_shape=jax.ShapeDtypeStruct((E, H), jnp.float32),
    )(g0, We2, be2)
    mij = pl.pallas_call(
        _edge_body_acc,
        grid=(nblk,),
        in_specs=[
            pl.BlockSpec(memory_space=pltpu.HBM),
            pl.BlockSpec((BE, H), lambda i: (i, 0)),
            pl.BlockSpec((H, H), lambda i: (0, 0)),
            pl.BlockSpec((H,), lambda i: (0,)),
        ],
        out_specs=pl.BlockSpec((BE, H), lambda i, _n=nblk: (i + _n, 0)),
        out_shape=jax.ShapeDtypeStruct((E, H), jnp.float32),
        input_output_aliases={0: 0},
    )(mij0, g1, We2, be2)

    partials = _make_scatter_add(N, H, E)(mij, row)

    h_new = pl.pallas_call(
        _node_body,
        grid=(grid_n,),
        in_specs=[
            pl.BlockSpec((BN, D), lambda i: (i, 0)),
            pl.BlockSpec((_NC, BN, H), lambda i: (0, i, 0)),
            pl.BlockSpec((2 * D, H), lambda i: (0, 0)),
            pl.BlockSpec((H,), lambda i: (0,)),
            pl.BlockSpec((H, D), lambda i: (0, 0)),
            pl.BlockSpec((D,), lambda i: (0,)),
        ],
        out_specs=pl.BlockSpec((BN, D), lambda i: (i, 0)),
        out_shape=jax.ShapeDtypeStruct((N, D), jnp.float32),
    )(h_q, partials, Wn1, bn1, Wn2, bn2)

    return (h_new, mij)


# speedup vs baseline: 1.7048x; 1.0112x over previous
import functools

import jax
import jax.numpy as jnp
from jax import lax
from jax.experimental import pallas as pl
from jax.experimental.pallas import tpu as pltpu
from jax.experimental.pallas import tpu_sc as plsc

_NC = 2
_NS = 16
_NW = _NC * _NS
_L = 16


def _silu(x):
    return x * lax.logistic(x)


def _split_body(e_ref, row_ref, col_ref):
    row_ref[...] = e_ref[0]
    col_ref[...] = e_ref[1]


def _proj_body(hq_ref, hkv_ref, w1_ref, be1_ref, aq_ref, ak_ref):
    d = hq_ref.shape[1]
    aq_ref[...] = jnp.dot(hq_ref[...], w1_ref[:d],
                          preferred_element_type=jnp.float32)
    ak_ref[...] = jnp.dot(hkv_ref[...], w1_ref[d:],
                          preferred_element_type=jnp.float32) + be1_ref[...]


_NBUF = 4
_GBUF = 5
_GLEAD = 3


@functools.lru_cache(maxsize=None)
def _make_gather_add(N, H, esz, eoff, C):
    epw = esz // _NW
    nch = epw // C
    ngrp = nch // _GBUF
    assert nch == ngrp * _GBUF
    mesh = plsc.VectorSubcoreMesh(core_axis_name="c", subcore_axis_name="s")

    @functools.partial(
        pl.kernel,
        out_type=jax.ShapeDtypeStruct((esz, H), jnp.float32),
        mesh=mesh,
        scratch_types=[
            pltpu.VMEM((epw,), jnp.int32),
            pltpu.VMEM((epw,), jnp.int32),
            [pltpu.VMEM((C, H), jnp.float32)] * _GBUF,
            [pltpu.VMEM((C, H), jnp.float32)] * _GBUF,
            [pltpu.SemaphoreType.DMA] * _GBUF,
            [pltpu.SemaphoreType.DMA] * _GBUF,
        ],
    )
    def gather_add(aq_hbm, ak_hbm, row_hbm, col_hbm, out_hbm,
                   ridx, cidx, bqs, bks, gsems, osems):
        wid = lax.axis_index("s") * _NC + lax.axis_index("c")
        base = wid * epw

        pltpu.sync_copy(row_hbm.at[pl.ds(eoff + base, epw)], ridx)
        pltpu.sync_copy(col_hbm.at[pl.ds(eoff + base, epw)], cidx)

        def issue_gather(k, b):
            pltpu.async_copy(aq_hbm.at[ridx.at[pl.ds(k * C, C)]], bqs[b],
                             gsems[b])
            pltpu.async_copy(ak_hbm.at[cidx.at[pl.ds(k * C, C)]], bks[b],
                             gsems[b])

        def wait_gather(k, b):
            pltpu.make_async_copy(aq_hbm.at[ridx.at[pl.ds(k * C, C)]],
                                  bqs[b], gsems[b]).wait()
            pltpu.make_async_copy(ak_hbm.at[cidx.at[pl.ds(k * C, C)]],
                                  bks[b], gsems[b]).wait()

        def add_and_out(k, b):
            bq, bk = bqs[b], bks[b]

            def add_row(e, c2):
                for j in range(H // _L):
                    sl = pl.ds(j * _L, _L)
                    plsc.addupdate(bq.at[e, sl], bk[e, sl])
                return c2

            lax.fori_loop(0, C, add_row, 0)
            pltpu.async_copy(bq, out_hbm.at[pl.ds(base + k * C, C)],
                             osems[b])

        def wait_out(b):
            pltpu.make_async_copy(bqs[b], out_hbm.at[pl.ds(base, C)],
                                  osems[b]).wait()

        for k0 in range(_GLEAD):
            issue_gather(k0, k0)

        def group(i, c):
            for p in range(_GBUF):
                k = _GBUF * i + p
                nk = k + _GLEAD
                nb = (p + _GLEAD) % _GBUF
                if p < _GLEAD - 1:
                    @pl.when(i > 0)
                    def _():
                        wait_out(nb)
                    issue_gather(nk, nb)
                else:
                    @pl.when(i < ngrp - 1)
                    def _():
                        wait_out(nb)
                        issue_gather(nk, nb)
                wait_gather(k, p)
                add_and_out(k, p)
            return c

        lax.fori_loop(0, ngrp, group, 0)
        for b in range(_GBUF):
            wait_out(b)

    return gather_add


def _edge_body(g_ref, w2_ref, b2_ref, out_ref):
    h1 = _silu(g_ref[...])
    out_ref[...] = _silu(jnp.dot(h1, w2_ref[...],
                                 preferred_element_type=jnp.float32)
                         + b2_ref[...])


def _edge_body_acc(prev_ref, g_ref, w2_ref, b2_ref, out_ref):
    del prev_ref
    _edge_body(g_ref, w2_ref, b2_ref, out_ref)


@functools.lru_cache(maxsize=None)
def _make_scatter_add(N, H, E):
    epw = E // _NW
    C = 80
    nch = epw // C
    n_pad = ((N + _NS * 8 - 1) // (_NS * 8)) * (_NS * 8)
    rpt = n_pad // _NS
    mesh = plsc.VectorSubcoreMesh(core_axis_name="c", subcore_axis_name="s")

    ngrp = (nch - 1) // _NBUF
    assert nch == 1 + ngrp * _NBUF

    @functools.partial(
        pl.kernel,
        out_type=jax.ShapeDtypeStruct((_NC, n_pad, H), jnp.float32),
        mesh=mesh,
        scratch_types=[
            [pltpu.VMEM((C,), jnp.int32)] * _NBUF,
            [pltpu.VMEM((C, H), jnp.float32)] * _NBUF,
            pltpu.VMEM_SHARED((n_pad, H), jnp.float32),
            [pltpu.SemaphoreType.DMA] * _NBUF,
            [pltpu.SemaphoreType.DMA] * _NBUF,
        ],
    )
    def scatter_add(mij_hbm, row_hbm, out_hbm, idxs, vals, acc,
                    vsems, ssems):
        cid = lax.axis_index("c")
        sid = lax.axis_index("s")
        wid = sid * _NC + cid
        base = wid * epw

        zero = jnp.zeros((_L,), jnp.float32)

        def zrow(r, c2):
            for j in range(H // _L):
                vals[0][r, pl.ds(j * _L, _L)] = zero
            return c2

        lax.fori_loop(0, C, zrow, 0)
        nz = (rpt + C - 1) // C

        def zcopy(t, c2):
            off = jnp.minimum(t * C, rpt - C)
            pltpu.sync_copy(vals[0], acc.at[pl.ds(sid * rpt + off, C)])
            return c2

        lax.fori_loop(0, nz, zcopy, 0)
        plsc.subcore_barrier()

        def issue_copies(k, b):
            off = base + k * C
            pltpu.async_copy(row_hbm.at[pl.ds(off, C)], idxs[b], vsems[b])
            pltpu.async_copy(mij_hbm.at[pl.ds(off, C)], vals[b], vsems[b])

        def wait_copies(k, b):
            off = base + k * C
            pltpu.make_async_copy(row_hbm.at[pl.ds(off, C)], idxs[b],
                                  vsems[b]).wait()
            pltpu.make_async_copy(mij_hbm.at[pl.ds(off, C)], vals[b],
                                  vsems[b]).wait()

        def issue_scatter(b):
            pltpu.async_copy(vals[b], acc.at[idxs[b]], ssems[b], add=True)

        def wait_scatter(b):
            pltpu.make_async_copy(vals[b], acc.at[idxs[b]],
                                  ssems[b]).wait()

        issue_copies(0, 0)
        issue_copies(1, 1)
        wait_copies(0, 0)
        issue_scatter(0)

        def group(i, c):
            for p in range(_NBUF):
                k = _NBUF * i + 1 + p
                b = (p + 1) % _NBUF
                nk = k + 1
                nb = (b + 1) % _NBUF
                if p == 2:
                    wait_scatter(nb)
                    issue_copies(nk, nb)
                elif p == 0 or p == 1:
                    @pl.when(i > 0)
                    def _():
                        wait_scatter(nb)
                    issue_copies(nk, nb)
                else:
                    @pl.when(i < ngrp - 1)
                    def _():
                        wait_scatter(nb)
                        issue_copies(nk, nb)
                wait_copies(k, b)
                issue_scatter(b)
            return c

        lax.fori_loop(0, ngrp, group, 0)
        for b in range(_NBUF):
            wait_scatter(b)
        plsc.subcore_barrier()

        pltpu.sync_copy(acc.at[pl.ds(sid * rpt, rpt)],
                        out_hbm.at[cid, pl.ds(sid * rpt, rpt)])

    return scatter_add


def _node_body(hq_ref, p_ref, wn1_ref, bn1_ref, wn2_ref, bn2_ref,
               out_ref):
    agg = p_ref[0] + p_ref[1]
    hq = hq_ref[...]
    d = hq_ref.shape[1]
    t = _silu(jnp.dot(hq, wn1_ref[:d], preferred_element_type=jnp.float32)
              + jnp.dot(agg, wn1_ref[d:], preferred_element_type=jnp.float32)
              + bn1_ref[...])
    out_ref[...] = hq + jnp.dot(t, wn2_ref[...],
                                preferred_element_type=jnp.float32) + bn2_ref[...]


def kernel(h_q, h_kv, edge_index, We1, be1, We2, be2, Wn1, bn1, Wn2, bn2):
    N, D = h_q.shape
    H = We2.shape[0]
    E = edge_index.shape[1]

    row, col = pl.pallas_call(
        _split_body,
        out_shape=[jax.ShapeDtypeStruct((E,), jnp.int32)] * 2,
    )(edge_index)

    BN = 2000
    grid_n = N // BN
    aq, ak = pl.pallas_call(
        _proj_body,
        grid=(grid_n,),
        in_specs=[
            pl.BlockSpec((BN, D), lambda i: (i, 0)),
            pl.BlockSpec((BN, D), lambda i: (i, 0)),
            pl.BlockSpec((2 * D, H), lambda i: (0, 0)),
            pl.BlockSpec((H,), lambda i: (0,)),
        ],
        out_specs=[
            pl.BlockSpec((BN, H), lambda i: (i, 0)),
            pl.BlockSpec((BN, H), lambda i: (i, 0)),
        ],
        out_shape=[jax.ShapeDtypeStruct((N, H), jnp.float32)] * 2,
    )(h_q, h_kv, We1, be1)

    E2 = E // 2
    g0 = _make_gather_add(N, H, E2, 0, 40)(aq, ak, row, col)
    g1 = _make_gather_add(N, H, E2, E2, 40)(aq, ak, row, col)

    BE = 16000
    nblk = E2 // BE
    mij0 = pl.pallas_call(
        _edge_body,
        grid=(nblk,),
        in_specs=[
            pl.BlockSpec((BE, H), lambda i: (i, 0)),
            pl.BlockSpec((H, H), lambda i: (0, 0)),
            pl.BlockSpec((H,), lambda i: (0,)),
        ],
        out_specs=pl.BlockSpec((BE, H), lambda i: (i, 0)),
        out_shape=jax.ShapeDtypeStruct((E, H), jnp.float32),
    )(g0, We2, be2)
    mij = pl.pallas_call(
        _edge_body_acc,
        grid=(nblk,),
        in_specs=[
            pl.BlockSpec(memory_space=pltpu.HBM),
            pl.BlockSpec((BE, H), lambda i: (i, 0)),
            pl.BlockSpec((H, H), lambda i: (0, 0)),
            pl.BlockSpec((H,), lambda i: (0,)),
        ],
        out_specs=pl.BlockSpec((BE, H), lambda i, _n=nblk: (i + _n, 0)),
        out_shape=jax.ShapeDtypeStruct((E, H), jnp.float32),
        input_output_aliases={0: 0},
    )(mij0, g1, We2, be2)

    partials = _make_scatter_add(N, H, E)(mij, row)

    h_new = pl.pallas_call(
        _node_body,
        grid=(grid_n,),
        in_specs=[
            pl.BlockSpec((BN, D), lambda i: (i, 0)),
            pl.BlockSpec((_NC, BN, H), lambda i: (0, i, 0)),
            pl.BlockSpec((2 * D, H), lambda i: (0, 0)),
            pl.BlockSpec((H,), lambda i: (0,)),
            pl.BlockSpec((H, D), lambda i: (0, 0)),
            pl.BlockSpec((D,), lambda i: (0,)),
        ],
        out_specs=pl.BlockSpec((BN, D), lambda i: (i, 0)),
        out_shape=jax.ShapeDtypeStruct((N, D), jnp.float32),
    )(h_q, partials, Wn1, bn1, Wn2, bn2)

    return (h_new, mij)
